# baseline (speedup 1.0000x reference)
import functools

import jax
import jax.numpy as jnp
from jax import lax
from jax.experimental import pallas as pl
from jax.experimental.pallas import tpu as pltpu
from jax.experimental.pallas import tpu_sc as plsc

B, S, LANES = 4, 8192, 2048
NW = 32
LC = LANES * B // NW
NCHUNK = LC // 16
ST = 64
NTILES = S // ST
NBUF = 2
NGROUPS = NTILES // NBUF

_mesh = plsc.VectorSubcoreMesh(core_axis_name="c", subcore_axis_name="s")


@functools.partial(
    pl.kernel,
    out_type=jax.ShapeDtypeStruct((B, S, LANES), jnp.float32),
    mesh=_mesh,
    scratch_types=[
        pltpu.VMEM((NBUF, ST, LC), jnp.float32),
        pltpu.VMEM((NBUF, ST, LC), jnp.float32),
        [pltpu.SemaphoreType.DMA] * NBUF,
        [pltpu.SemaphoreType.DMA] * NBUF,
    ],
)
def _cumsum_sc(x_hbm, out_hbm, inbuf, outbuf, insems, outsems):
    wid = lax.axis_index("s") * 2 + lax.axis_index("c")
    b = wid // (NW // B)
    l0 = (wid % (NW // B)) * LC

    def in_copy(t, slot):
        src = x_hbm.at[b, pl.ds(t * ST, ST), pl.ds(l0, LC)]
        return pltpu.make_async_copy(src, inbuf.at[slot], insems[slot])

    def out_copy(t, slot):
        dst = out_hbm.at[b, pl.ds(t * ST, ST), pl.ds(l0, LC)]
        return pltpu.make_async_copy(outbuf.at[slot], dst, outsems[slot])

    for slot in range(NBUF):
        in_copy(slot, slot).start()

    def group_body(g, carries):
        for slot in range(NBUF):
            t = NBUF * g + slot
            in_copy(t, slot).wait()

            @pl.when(g >= 1)
            def _wait_prev_out(slot=slot, t=t):
                out_copy(t - NBUF, slot).wait()

            def row_body(r, cs, slot=slot):
                new = []
                for j in range(NCHUNK):
                    c = cs[j] + inbuf[slot, r, pl.ds(j * 16, 16)]
                    outbuf[slot, r, pl.ds(j * 16, 16)] = c
                    new.append(c)
                return tuple(new)

            carries = lax.fori_loop(0, ST, row_body, carries, unroll=8)
            out_copy(t, slot).start()

            @pl.when(g + 1 < NGROUPS)
            def _prefetch(slot=slot, t=t):
                in_copy(t + NBUF, slot).start()

        return carries

    lax.fori_loop(0, NGROUPS, group_body,
                  tuple(jnp.zeros((16,), jnp.float32) for _ in range(NCHUNK)))

    for slot in range(NBUF):
        out_copy(NTILES - NBUF + slot, slot).wait()


def kernel(masks):
    return _cumsum_sc(masks)


# speedup vs baseline: 1.4922x; 1.4922x over previous
import functools

import jax
import jax.numpy as jnp
from jax import lax
from jax.experimental import pallas as pl
from jax.experimental.pallas import tpu as pltpu
from jax.experimental.pallas import tpu_sc as plsc

B, S, LANES = 4, 8192, 2048
NW = 32
LC = LANES * B // NW
NCHUNK = LC // 16
ST = 64
NTILES = S // ST
NBUF = 2
NGROUPS = NTILES // NBUF

_mesh = plsc.VectorSubcoreMesh(core_axis_name="c", subcore_axis_name="s")


@functools.partial(
    pl.kernel,
    out_type=jax.ShapeDtypeStruct((B, S, LANES), jnp.float32),
    mesh=_mesh,
    scratch_types=[
        pltpu.VMEM((NBUF, ST, LC), jnp.float32),
        pltpu.VMEM((NBUF, ST, LC), jnp.float32),
        [pltpu.SemaphoreType.DMA] * NBUF,
        [pltpu.SemaphoreType.DMA] * NBUF,
    ],
)
def _cumsum_sc(x_hbm, out_hbm, inbuf, outbuf, insems, outsems):
    wid = lax.axis_index("s") * 2 + lax.axis_index("c")
    b = wid // (NW // B)
    l0 = (wid % (NW // B)) * LC

    def in_copy(t, slot):
        src = x_hbm.at[b, pl.ds(t * ST, ST), pl.ds(l0, LC)]
        return pltpu.make_async_copy(src, inbuf.at[slot], insems[slot])

    def out_copy(t, slot):
        dst = out_hbm.at[b, pl.ds(t * ST, ST), pl.ds(l0, LC)]
        return pltpu.make_async_copy(outbuf.at[slot], dst, outsems[slot])

    for slot in range(NBUF):
        in_copy(slot, slot).start()

    def group_body(g, carries):
        for slot in range(NBUF):
            t = NBUF * g + slot
            in_copy(t, slot).wait()

            @pl.when(g >= 1)
            def _wait_prev_out(slot=slot, t=t):
                out_copy(t - NBUF, slot).wait()

            def row_body(r, cs, slot=slot):
                new = []
                for j in range(NCHUNK):
                    c = cs[j] + inbuf[slot, r, pl.ds(j * 16, 16)]
                    outbuf[slot, r, pl.ds(j * 16, 16)] = c
                    new.append(c)
                return tuple(new)

            carries = lax.fori_loop(0, ST, row_body, carries, unroll=4)
            out_copy(t, slot).start()

            @pl.when(g + 1 < NGROUPS)
            def _prefetch(slot=slot, t=t):
                in_copy(t + NBUF, slot).start()

        return carries

    lax.fori_loop(0, NGROUPS, group_body,
                  tuple(jnp.zeros((16,), jnp.float32) for _ in range(NCHUNK)))

    for slot in range(NBUF):
        out_copy(NTILES - NBUF + slot, slot).wait()


def kernel(masks):
    return _cumsum_sc(masks)
